# baseline (speedup 1.0000x reference)
import functools

import jax
import jax.numpy as jnp
from jax import lax
from jax.experimental import pallas as pl
from jax.experimental.pallas import tpu as pltpu
from jax.experimental.pallas import tpu_sc as plsc

N = 4
VOCAB = 1000000
D = 16
TOT = 4096 * 50
NC = 2
NS = 16
NW = NC * NS
BPW = TOT // NW
CH = 128
NCH = BPW // CH
LANES = 16
SCALE = 4.0


def _body(x_hbm, w_hbm, out_hbm, idx_v, widx_v, gout_v, rows_v, sem):
  wid = lax.axis_index("s") * NC + lax.axis_index("c")
  base = wid * BPW
  pltpu.sync_copy(x_hbm.at[wid], idx_v)
  dlanes = lax.iota(jnp.int32, LANES)

  def table_body(t, carry):
    def chunk_body(j, carry2):
      for k in range(CH // LANES):
        sl = pl.ds(k * LANES, LANES)
        iv = idx_v[j, sl]
        for d in range(D):
          widx_v[d, sl] = iv + (t * D + d) * VOCAB
      descs = []
      for r in range(D):
        descs.append(
            pltpu.async_copy(w_hbm.at[widx_v.at[r]], gout_v.at[r], sem))
      for de in descs:
        de.wait()
      for i in range(CH):
        col = jnp.full((LANES,), i, jnp.int32)
        rows_v[pl.ds(i * D, D)] = (
            plsc.load_gather(gout_v, [dlanes, col]) * SCALE)
      dst = (t * TOT + base + j * CH) * D
      pltpu.sync_copy(rows_v, out_hbm.at[pl.ds(dst, CH * D)])
      return carry2

    lax.fori_loop(0, NCH, chunk_body, 0)
    return carry

  lax.fori_loop(0, N, table_body, 0)


def kernel(x, W):
  xr = x.reshape(NW, NCH, CH).astype(jnp.int32)
  wd = W.transpose(0, 2, 1).reshape(N * D * VOCAB)
  mesh = plsc.VectorSubcoreMesh(
      core_axis_name="c", subcore_axis_name="s", num_cores=NC,
      num_subcores=NS)
  call = pl.kernel(
      _body,
      out_type=jax.ShapeDtypeStruct((N * TOT * D,), jnp.float32),
      mesh=mesh,
      scratch_types=[
          pltpu.VMEM((NCH, CH), jnp.int32),
          pltpu.VMEM((D, CH), jnp.int32),
          pltpu.VMEM((D, CH), jnp.float32),
          pltpu.VMEM((CH * D,), jnp.float32),
          pltpu.SemaphoreType.DMA,
      ],
      compiler_params=pltpu.CompilerParams(
          use_tc_tiling_on_sc=False, needs_layout_passes=False),
  )
  out = call(xr, wd)
  b, l = x.shape
  out = out.reshape(N, TOT, D)
  return tuple(out[i].reshape(b, l, D) for i in range(N))


# speedup vs baseline: 2.7657x; 2.7657x over previous
import functools

import jax
import jax.numpy as jnp
from jax import lax
from jax.experimental import pallas as pl
from jax.experimental.pallas import tpu as pltpu
from jax.experimental.pallas import tpu_sc as plsc

N = 4
VOCAB = 1000000
D = 16
DPAD = 128
RPT = DPAD // D
TOT = 4096 * 50
NC = 2
NS = 16
NW = NC * NS
BPW = TOT // NW
CH = 128
NCH = BPW // CH
LANES = 16
SCALE = 4.0


def _body(x_hbm, w_hbm, o0, o1, o2, o3, idx_v, widx_v, rows_v, sem):
  wid = lax.axis_index("s") * NC + lax.axis_index("c")
  base = wid * BPW
  pltpu.sync_copy(x_hbm.at[wid], idx_v)

  for t, out_ref in enumerate((o0, o1, o2, o3)):

    def chunk_body(j, carry, t=t, out_ref=out_ref):
      for k in range(CH // LANES):
        sl = pl.ds(k * LANES, LANES)
        widx_v[sl] = (idx_v[j, sl] + t * VOCAB) * RPT
      pltpu.async_copy(w_hbm.at[widx_v], rows_v, sem).wait()
      for i in range(CH):
        rows_v[i] = rows_v[i] * SCALE
      pltpu.sync_copy(rows_v, out_ref.at[pl.ds(base + j * CH, CH)])
      return carry

    lax.fori_loop(0, NCH, chunk_body, 0)


def kernel(x, W):
  xr = x.reshape(NW, NCH, CH).astype(jnp.int32)
  wp = jnp.pad(W, ((0, 0), (0, 0), (0, DPAD - D)))
  wf = wp.reshape(N * VOCAB * RPT, D)
  mesh = plsc.VectorSubcoreMesh(
      core_axis_name="c", subcore_axis_name="s", num_cores=NC,
      num_subcores=NS)
  out_struct = jax.ShapeDtypeStruct((TOT, D), jnp.float32)
  call = pl.kernel(
      _body,
      out_type=(out_struct,) * N,
      mesh=mesh,
      scratch_types=[
          pltpu.VMEM((NCH, CH), jnp.int32),
          pltpu.VMEM((CH,), jnp.int32),
          pltpu.VMEM((CH, D), jnp.float32),
          pltpu.SemaphoreType.DMA,
      ],
      compiler_params=pltpu.CompilerParams(use_tc_tiling_on_sc=False),
  )
  outs = call(xr, wf)
  b, l = x.shape
  return tuple(o.reshape(b, l, D) for o in outs)


# speedup vs baseline: 3.7232x; 1.3462x over previous
import functools

import jax
import jax.numpy as jnp
from jax import lax
from jax.experimental import pallas as pl
from jax.experimental.pallas import tpu as pltpu
from jax.experimental.pallas import tpu_sc as plsc

N = 4
VOCAB = 1000000
D = 16
DPAD = 128
RPT = DPAD // D
TOT = 4096 * 50
NC = 2
NS = 16
NW = NC * NS
BPW = TOT // NW
CH = 128
NCH = BPW // CH
LANES = 16
SCALE = 4.0


def _body(x_hbm, w_hbm, o0, o1, o2, o3, idx_v, widx_v, rows_v, sem):
  wid = lax.axis_index("s") * NC + lax.axis_index("c")
  base = wid * BPW
  pltpu.sync_copy(x_hbm.at[wid], idx_v)

  for t, out_ref in enumerate((o0, o1, o2, o3)):

    def chunk_body(j, carry, t=t, out_ref=out_ref):
      for k in range(CH // LANES):
        sl = pl.ds(k * LANES, LANES)
        iv = idx_v[j, sl]
        widx_v[sl] = (
            (iv & -4096) + ((iv & 511) << 3) + ((iv >> 9) & 7) + t * TSTRIDE)
      pltpu.async_copy(w_hbm.at[widx_v], rows_v, sem).wait()
      for i in range(CH):
        rows_v[i] = rows_v[i] * SCALE
      pltpu.sync_copy(rows_v, out_ref.at[pl.ds(base + j * CH, CH)])
      return carry

    lax.fori_loop(0, NCH, chunk_body, 0)


BLK = 4096
SUB = BLK // 8
NBLK = -(-VOCAB // BLK)
TSTRIDE = NBLK * BLK


def _detile_body(wt_ref, o_ref):
  v = wt_ref[0]
  for k in range(8):
    o_ref[0, :, k * D:(k + 1) * D] = v[:, k * SUB:(k + 1) * SUB].T


def _detile(W):
  wt = W.transpose(0, 2, 1)
  out = pl.pallas_call(
      _detile_body,
      grid=(N, NBLK),
      in_specs=[pl.BlockSpec((1, D, BLK), lambda t, j: (t, 0, j))],
      out_specs=pl.BlockSpec((1, SUB, 128), lambda t, j: (t, j, 0)),
      out_shape=jax.ShapeDtypeStruct((N, NBLK * SUB, 128), jnp.float32),
  )(wt)
  return out.reshape(N * TSTRIDE, D)


def kernel(x, W):
  xr = x.reshape(NW, NCH, CH).astype(jnp.int32)
  wf = _detile(W)
  mesh = plsc.VectorSubcoreMesh(
      core_axis_name="c", subcore_axis_name="s", num_cores=NC,
      num_subcores=NS)
  out_struct = jax.ShapeDtypeStruct((TOT, D), jnp.float32)
  call = pl.kernel(
      _body,
      out_type=(out_struct,) * N,
      mesh=mesh,
      scratch_types=[
          pltpu.VMEM((NCH, CH), jnp.int32),
          pltpu.VMEM((CH,), jnp.int32),
          pltpu.VMEM((CH, D), jnp.float32),
          pltpu.SemaphoreType.DMA,
      ],
      compiler_params=pltpu.CompilerParams(use_tc_tiling_on_sc=False),
  )
  outs = call(xr, wf)
  b, l = x.shape
  return tuple(o.reshape(b, l, D) for o in outs)


# speedup vs baseline: 3.8336x; 1.0296x over previous
import functools

import jax
import jax.numpy as jnp
from jax import lax
from jax.experimental import pallas as pl
from jax.experimental.pallas import tpu as pltpu
from jax.experimental.pallas import tpu_sc as plsc

N = 4
VOCAB = 1000000
D = 16
DPAD = 128
RPT = DPAD // D
TOT = 4096 * 50
NC = 2
NS = 16
NW = NC * NS
BPW = TOT // NW
CH = 128
NCH = BPW // CH
LANES = 16
SCALE = 4.0


def _body(x_hbm, w_hbm, o0, o1, o2, o3, idx_v, widx_v, rows_v, sem):
  wid = lax.axis_index("s") * NC + lax.axis_index("c")
  base = wid * BPW
  pltpu.sync_copy(x_hbm.at[wid], idx_v)

  for t, out_ref in enumerate((o0, o1, o2, o3)):

    def chunk_body(j, carry, t=t, out_ref=out_ref):
      for k in range(CH // LANES):
        sl = pl.ds(k * LANES, LANES)
        iv = idx_v[j, sl]
        widx_v[sl] = (
            (iv & -8192) + ((iv & 1023) << 3) + ((iv >> 10) & 7) + t * TSTRIDE)
      pltpu.async_copy(w_hbm.at[widx_v], rows_v, sem).wait()
      for i in range(CH):
        rows_v[i] = rows_v[i] * SCALE
      pltpu.sync_copy(rows_v, out_ref.at[pl.ds(base + j * CH, CH)])
      return carry

    lax.fori_loop(0, NCH, chunk_body, 0)


BLK = 8192
SUB = BLK // 8
NBLK = -(-VOCAB // BLK)
TSTRIDE = NBLK * BLK


def _detile_body(wt_ref, o_ref):
  v = wt_ref[0]
  for k in range(8):
    o_ref[0, :, k * D:(k + 1) * D] = v[:, k * SUB:(k + 1) * SUB].T


def _detile(W):
  wt = W.transpose(0, 2, 1)
  out = pl.pallas_call(
      _detile_body,
      grid=(N, NBLK),
      in_specs=[pl.BlockSpec((1, D, BLK), lambda t, j: (t, 0, j))],
      out_specs=pl.BlockSpec((1, SUB, 128), lambda t, j: (t, j, 0)),
      out_shape=jax.ShapeDtypeStruct((N, NBLK * SUB, 128), jnp.float32),
  )(wt)
  return out.reshape(N * TSTRIDE, D)


def kernel(x, W):
  xr = x.reshape(NW, NCH, CH).astype(jnp.int32)
  wf = _detile(W)
  mesh = plsc.VectorSubcoreMesh(
      core_axis_name="c", subcore_axis_name="s", num_cores=NC,
      num_subcores=NS)
  out_struct = jax.ShapeDtypeStruct((TOT, D), jnp.float32)
  call = pl.kernel(
      _body,
      out_type=(out_struct,) * N,
      mesh=mesh,
      scratch_types=[
          pltpu.VMEM((NCH, CH), jnp.int32),
          pltpu.VMEM((CH,), jnp.int32),
          pltpu.VMEM((CH, D), jnp.float32),
          pltpu.SemaphoreType.DMA,
      ],
      compiler_params=pltpu.CompilerParams(use_tc_tiling_on_sc=False),
  )
  outs = call(xr, wf)
  b, l = x.shape
  return tuple(o.reshape(b, l, D) for o in outs)


# speedup vs baseline: 3.8362x; 1.0007x over previous
import functools

import jax
import jax.numpy as jnp
from jax import lax
from jax.experimental import pallas as pl
from jax.experimental.pallas import tpu as pltpu
from jax.experimental.pallas import tpu_sc as plsc

N = 4
VOCAB = 1000000
D = 16
TOT = 4096 * 50
NC = 2
NS = 16
NW = NC * NS
BPW = TOT // NW
CH = 128
NCH = BPW // CH
LANES = 16
SCALE = 4.0


def _body(x_hbm, w_hbm, o0, o1, o2, o3, idx_v, widx_v, rows_v, sem):
  wid = lax.axis_index("s") * NC + lax.axis_index("c")
  base = wid * BPW
  pltpu.sync_copy(x_hbm.at[wid], idx_v)

  for t, out_ref in enumerate((o0, o1, o2, o3)):

    def chunk_body(j, carry, t=t, out_ref=out_ref):
      for k in range(CH // LANES):
        sl = pl.ds(k * LANES, LANES)
        iv = idx_v[j, sl]
        widx_v[sl] = (
            (iv & -8192) + ((iv & 1023) << 3) + ((iv >> 10) & 7) + t * TSTRIDE)
      pltpu.async_copy(w_hbm.at[widx_v], rows_v, sem).wait()
      for i in range(CH):
        rows_v[i] = rows_v[i] * SCALE
      pltpu.sync_copy(rows_v, out_ref.at[pl.ds(base + j * CH, CH)])
      return carry

    lax.fori_loop(0, NCH, chunk_body, 0)


BLK = 8192
SUB = BLK // 8
NBLK = -(-VOCAB // BLK)
TSTRIDE = NBLK * BLK


def _detile_body(wt_ref, o_ref):
  v = wt_ref[0]
  for k in range(8):
    o_ref[0, :, k * D:(k + 1) * D] = v[:, k * SUB:(k + 1) * SUB].T


def _detile(W):
  wt = W.transpose(0, 2, 1)
  out = pl.pallas_call(
      _detile_body,
      grid=(N, NBLK),
      in_specs=[pl.BlockSpec((1, D, BLK), lambda t, j: (t, 0, j))],
      out_specs=pl.BlockSpec((1, SUB, 128), lambda t, j: (t, j, 0)),
      out_shape=jax.ShapeDtypeStruct((N, NBLK * SUB, 128), jnp.float32),
  )(wt)
  return out.reshape(N * TSTRIDE, D)


def kernel(x, W):
  xr = x.reshape(NW, NCH, CH).astype(jnp.int32)
  wf = _detile(W)
  mesh = plsc.VectorSubcoreMesh(
      core_axis_name="c", subcore_axis_name="s", num_cores=NC,
      num_subcores=NS)
  out_struct = jax.ShapeDtypeStruct((TOT, D), jnp.float32)
  call = pl.kernel(
      _body,
      out_type=(out_struct,) * N,
      mesh=mesh,
      scratch_types=[
          pltpu.VMEM((NCH, CH), jnp.int32),
          pltpu.VMEM((CH,), jnp.int32),
          pltpu.VMEM((CH, D), jnp.float32),
          pltpu.SemaphoreType.DMA,
      ],
      compiler_params=pltpu.CompilerParams(use_tc_tiling_on_sc=False),
  )
  outs = call(xr, wf)
  b, l = x.shape
  return tuple(o.reshape(b, l, D) for o in outs)
